# Initial kernel scaffold; baseline (speedup 1.0000x reference)
import jax
import jax.numpy as jnp
from jax.experimental import pallas as pl


def kernel(features, W_conv, b_conv, W_cls, b_cls, W_bbox, b_bbox):
    raise NotImplementedError("write your pallas kernel here")


# speedup vs baseline: 3.0901x; 3.0901x over previous
import functools

import jax
import jax.numpy as jnp
import numpy as np
from jax import lax
from jax.experimental import pallas as pl
from jax.experimental.pallas import tpu as pltpu

_B, _C, _H, _W = 2, 256, 50, 76
_A = 3
_STRIDE = 16.0
_IMG_H, _IMG_W = 800.0, 1216.0
_SCALES = (128.0, 256.0, 512.0)
_PRE_NMS = 1000
_POST_NMS = 300
_NMS_THRESH = 0.7
_MIN_SIZE = 1e-3
_BBOX_CLIP = float(np.log(1000.0 / 16.0))

_WP = _W + 2
_HP = _H + 2
_J = 3968
_XCOLS = 4224
_NEG = float("-inf")

_INTERPRET = False


def _stage_a_body(x_ref, w2_ref, bc_ref, wh_ref, bh_ref, big_ref, scal_ref):
    x = x_ref[0]
    acc = jnp.zeros((_C, _J), jnp.float32)
    for dy in range(3):
        for dx in range(3):
            s = dy * _WP + dx
            acc += jnp.dot(w2_ref[dy * 3 + dx], x[:, s:s + _J],
                           preferred_element_type=jnp.float32)
    t = jnp.maximum(acc + bc_ref[:, :1], 0.0)
    o16 = jnp.dot(wh_ref[...], t, preferred_element_type=jnp.float32)
    o16 = o16 + bh_ref[:, :1]

    obj = o16[0:3, :]
    dxv = o16[3:6, :]
    dyv = o16[6:9, :]
    dwv = o16[9:12, :]
    dhv = o16[12:15, :]

    jj = lax.broadcasted_iota(jnp.int32, (3, _J), 1)
    aa = lax.broadcasted_iota(jnp.int32, (3, _J), 0)
    hh = (jj // _WP).astype(jnp.float32)
    ww = (jj % _WP).astype(jnp.float32)
    scale = jnp.where(aa == 0, _SCALES[0],
                      jnp.where(aa == 1, _SCALES[1], _SCALES[2]))
    ctrx = _STRIDE * ww
    ctry = _STRIDE * hh

    pcx = dxv * scale + ctrx
    pcy = dyv * scale + ctry
    pw = jnp.exp(jnp.minimum(dwv, _BBOX_CLIP)) * scale
    ph = jnp.exp(jnp.minimum(dhv, _BBOX_CLIP)) * scale
    x1 = jnp.clip(pcx - 0.5 * pw, 0.0, _IMG_W)
    y1 = jnp.clip(pcy - 0.5 * ph, 0.0, _IMG_H)
    x2 = jnp.clip(pcx + 0.5 * pw, 0.0, _IMG_W)
    y2 = jnp.clip(pcy + 0.5 * ph, 0.0, _IMG_H)

    garbage = (jj % _WP >= _W) | (jj >= _H * _WP)
    objm = jnp.where(garbage, _NEG, obj)
    sig = 1.0 / (1.0 + jnp.exp(-obj))
    valid = ((x2 - x1) >= _MIN_SIZE) & ((y2 - y1) >= _MIN_SIZE) & (sig >= 0.0)
    nms_score = jnp.where(valid & (~garbage), sig, _NEG)

    lo0 = jnp.min(jnp.where(garbage, jnp.inf, obj))
    hi0 = jnp.max(objm) + 1.0

    def bis(_, c):
        lo, hi = c
        mid = 0.5 * (lo + hi)
        cnt = jnp.sum(jnp.where(objm >= mid, 1.0, 0.0))
        ge = cnt >= float(_PRE_NMS)
        return jnp.where(ge, mid, lo), jnp.where(ge, hi, mid)

    v, _ = lax.fori_loop(0, 64, bis, (lo0, hi0))
    cnt_gt = jnp.sum(jnp.where(objm > v, 1.0, 0.0))
    quota = float(_PRE_NMS) - cnt_gt

    m = jnp.max(objm)
    fiota = aa * _J + jj
    gidx = jnp.min(jnp.where(objm == m, fiota, jnp.int32(2 ** 30)))
    gsel = fiota == gidx
    gx1 = jnp.sum(jnp.where(gsel, x1, 0.0))
    gy1 = jnp.sum(jnp.where(gsel, y1, 0.0))
    gx2 = jnp.sum(jnp.where(gsel, x2, 0.0))
    gy2 = jnp.sum(jnp.where(gsel, y2, 0.0))

    big_ref[0, 0] = objm
    big_ref[0, 1] = nms_score
    big_ref[0, 2] = x1
    big_ref[0, 3] = y1
    big_ref[0, 4] = x2
    big_ref[0, 5] = y2
    scal_ref[0, 0] = jnp.stack([v, quota, gx1, gy1, gx2, gy2, cnt_gt, 0.0])


def _stage_d_body(big_ref, scal_ref, out_ref):
    objm = big_ref[0, 0]
    nms_score = big_ref[0, 1]
    x1 = big_ref[0, 2]
    y1 = big_ref[0, 3]
    x2 = big_ref[0, 4]
    y2 = big_ref[0, 5]
    v = scal_ref[0, 0, 0]
    gx1 = scal_ref[0, 0, 2]
    gy1 = scal_ref[0, 0, 3]
    gx2 = scal_ref[0, 0, 4]
    gy2 = scal_ref[0, 0, 5]

    sel = objm >= v
    s0 = jnp.where(sel, nms_score, _NEG)
    areas = (x2 - x1) * (y2 - y1)

    jj = lax.broadcasted_iota(jnp.int32, (3, _J), 1)
    aa = lax.broadcasted_iota(jnp.int32, (3, _J), 0)
    fiota = aa * _J + jj
    oiota = lax.broadcasted_iota(jnp.int32, (8, 128), 1) + \
        128 * lax.broadcasted_iota(jnp.int32, (8, 128), 0)

    def body(i, c):
        s, ox1, oy1, ox2, oy2 = c
        bv = jnp.max(s)
        bidx = jnp.min(jnp.where(s == bv, fiota, jnp.int32(2 ** 30)))
        bsel = fiota == bidx
        bx1 = jnp.sum(jnp.where(bsel, x1, 0.0))
        by1 = jnp.sum(jnp.where(bsel, y1, 0.0))
        bx2 = jnp.sum(jnp.where(bsel, x2, 0.0))
        by2 = jnp.sum(jnp.where(bsel, y2, 0.0))
        barea = jnp.sum(jnp.where(bsel, areas, 0.0))
        xx1 = jnp.maximum(bx1, x1)
        yy1 = jnp.maximum(by1, y1)
        xx2 = jnp.minimum(bx2, x2)
        yy2 = jnp.minimum(by2, y2)
        inter = jnp.maximum(xx2 - xx1, 0.0) * jnp.maximum(yy2 - yy1, 0.0)
        iou = inter / (barea + areas - inter + 1e-9)
        s = jnp.where(iou > _NMS_THRESH, _NEG, s)
        s = jnp.where(bsel, _NEG, s)
        picked = bv > _NEG
        wx1 = jnp.where(picked, bx1, gx1)
        wy1 = jnp.where(picked, by1, gy1)
        wx2 = jnp.where(picked, bx2, gx2)
        wy2 = jnp.where(picked, by2, gy2)
        hit = (oiota == i).astype(jnp.float32)
        return (s, ox1 + hit * wx1, oy1 + hit * wy1,
                ox2 + hit * wx2, oy2 + hit * wy2)

    z = jnp.zeros((8, 128), jnp.float32)
    _, ox1, oy1, ox2, oy2 = lax.fori_loop(0, _POST_NMS, body,
                                          (s0, z, z, z, z))
    out_ref[0, 0] = ox1
    out_ref[0, 1] = oy1
    out_ref[0, 2] = ox2
    out_ref[0, 3] = oy2


def kernel(features, W_conv, b_conv, W_cls, b_cls, W_bbox, b_bbox):
    f32 = jnp.float32
    xpad = jnp.pad(features, ((0, 0), (0, 0), (1, 1), (1, 1)))
    xflat = xpad.reshape(_B, _C, _HP * _WP)
    xflat = jnp.pad(xflat, ((0, 0), (0, 0), (0, _XCOLS - _HP * _WP)))
    w2 = W_conv.transpose(2, 3, 0, 1).reshape(9, _C, _C).astype(f32)
    perm = [a * 4 + k for k in range(4) for a in range(_A)]
    wh = jnp.concatenate([
        W_cls.reshape(_A, _C),
        W_bbox.reshape(4 * _A, _C)[jnp.array(perm)],
        jnp.zeros((1, _C), f32),
    ], axis=0)
    bh = jnp.concatenate([
        b_cls, b_bbox[jnp.array(perm)], jnp.zeros((1,), f32)]).reshape(16, 1)
    bc = b_conv.reshape(_C, 1)

    big, scal = pl.pallas_call(
        _stage_a_body,
        grid=(_B,),
        in_specs=[
            pl.BlockSpec((1, _C, _XCOLS), lambda b: (b, 0, 0)),
            pl.BlockSpec((9, _C, _C), lambda b: (0, 0, 0)),
            pl.BlockSpec((_C, 1), lambda b: (0, 0)),
            pl.BlockSpec((16, _C), lambda b: (0, 0)),
            pl.BlockSpec((16, 1), lambda b: (0, 0)),
        ],
        out_specs=[
            pl.BlockSpec((1, 6, 3, _J), lambda b: (b, 0, 0, 0)),
            pl.BlockSpec((1, 1, 8), lambda b: (b, 0, 0)),
        ],
        out_shape=[
            jax.ShapeDtypeStruct((_B, 6, 3, _J), f32),
            jax.ShapeDtypeStruct((_B, 1, 8), f32),
        ],
        interpret=_INTERPRET,
    )(xflat, w2, bc, wh, bh)

    out = pl.pallas_call(
        _stage_d_body,
        grid=(_B,),
        in_specs=[
            pl.BlockSpec((1, 6, 3, _J), lambda b: (b, 0, 0, 0)),
            pl.BlockSpec((1, 1, 8), lambda b: (b, 0, 0)),
        ],
        out_specs=pl.BlockSpec((1, 4, 8, 128), lambda b: (b, 0, 0, 0)),
        out_shape=jax.ShapeDtypeStruct((_B, 4, 8, 128), f32),
        interpret=_INTERPRET,
    )(big, scal)

    boxes = out.reshape(_B, 4, 1024)[:, :, :_POST_NMS]
    return boxes.transpose(0, 2, 1)
